# baseline (speedup 1.0000x reference)
import functools

import jax
import jax.numpy as jnp
from jax import lax
from jax.experimental import pallas as pl
from jax.experimental.pallas import tpu as pltpu
from jax.experimental.pallas import tpu_sc as plsc

MARGIN = 1.0
C = 0.25
EPSILON = 0.001

_NC = 2
_NS = 16
_NW = _NC * _NS


def _sc_gather(ent, nrm, hyp, h, r, t, hc, rc, tc):
    B = h.shape[0]
    D = ent.shape[1]
    per = B // _NW

    mesh = plsc.VectorSubcoreMesh(core_axis_name="c", subcore_axis_name="s")
    out_type = [jax.ShapeDtypeStruct((B, D), jnp.float32)] * 8

    @functools.partial(
        pl.kernel,
        out_type=out_type,
        mesh=mesh,
        compiler_params=pltpu.CompilerParams(use_tc_tiling_on_sc=False),
        scratch_types=[
            pltpu.VMEM((per,), jnp.int32),
            pltpu.VMEM((per, D), jnp.float32),
            pltpu.SemaphoreType.DMA,
        ],
    )
    def gather_k(ent_h, nrm_h, hyp_h, h_h, r_h, t_h, hc_h, rc_h, tc_h,
                 o_eh, o_et, o_nr, o_hr, o_ehc, o_etc, o_nrc, o_hrc,
                 idx_v, rows_v, sem):
        wid = lax.axis_index("s") * _NC + lax.axis_index("c")
        base = wid * per
        jobs = (
            (h_h, ent_h, o_eh),
            (t_h, ent_h, o_et),
            (r_h, nrm_h, o_nr),
            (r_h, hyp_h, o_hr),
            (hc_h, ent_h, o_ehc),
            (tc_h, ent_h, o_etc),
            (rc_h, nrm_h, o_nrc),
            (rc_h, hyp_h, o_hrc),
        )
        for idx_h, tab_h, out_h in jobs:
            pltpu.sync_copy(idx_h.at[pl.ds(base, per)], idx_v)
            pltpu.async_copy(tab_h.at[idx_v], rows_v, sem).wait()
            pltpu.sync_copy(rows_v, out_h.at[pl.ds(base, per)])

    return gather_k(ent, nrm, hyp, h, r, t, hc, rc, tc)


def _tc_loss(eh, et, nr, hr, ehc, etc_, nrc, hrc):
    B, D = eh.shape
    grid = 16
    ch = B // grid

    def body(eh_r, et_r, nr_r, hr_r, ehc_r, etc_r, nrc_r, hrc_r, o_r):
        i = pl.program_id(0)

        @pl.when(i == 0)
        def _init():
            o_r[...] = jnp.zeros_like(o_r)

        def dist(hm, tm, nm, hym):
            nn = jnp.sum(nm * nm, axis=1, keepdims=True)
            den = jnp.maximum(jnp.sqrt(nn), 1e-12)
            nu = nm / den
            hh = hm - jnp.sum(hm * nu, axis=1, keepdims=True) * nu
            th = tm - jnp.sum(tm * nu, axis=1, keepdims=True) * nu
            dd = hh + hym - th
            return jnp.sqrt(jnp.sum(dd * dd, axis=1))

        pos = dist(eh_r[...], et_r[...], nr_r[...], hr_r[...])
        neg = dist(ehc_r[...], etc_r[...], nrc_r[...], hrc_r[...])
        rank = jnp.sum(jnp.maximum(pos - neg + MARGIN, 0.0))

        def orth(hym, nm):
            dot = jnp.sum(hym * nm, axis=1) ** 2
            nrm2 = jnp.sum(hym * hym, axis=1)
            return jnp.sum(jnp.maximum(dot / nrm2 - EPSILON**2, 0.0))

        og = orth(hr_r[...], nr_r[...]) + orth(hrc_r[...], nrc_r[...])

        def scl(x):
            return jnp.sum(jnp.maximum(jnp.sum(x * x, axis=1) - 1.0, 0.0))

        sc = scl(eh_r[...]) + scl(et_r[...]) + scl(ehc_r[...]) + scl(etc_r[...])

        row_i = lax.broadcasted_iota(jnp.int32, (8, 128), 0)
        lane_i = lax.broadcasted_iota(jnp.int32, (8, 128), 1)
        zero = jnp.zeros((8, 128), jnp.float32)
        contrib = (
            jnp.where((row_i == 0) & (lane_i == 0), rank, zero)
            + jnp.where((row_i == 0) & (lane_i == 1), og, zero)
            + jnp.where((row_i == 0) & (lane_i == 2), sc, zero)
        )
        o_r[...] += contrib

    in_spec = pl.BlockSpec((ch, D), lambda i: (i, 0))
    return pl.pallas_call(
        body,
        grid=(grid,),
        in_specs=[in_spec] * 8,
        out_specs=pl.BlockSpec((8, 128), lambda i: (0, 0)),
        out_shape=jax.ShapeDtypeStruct((8, 128), jnp.float32),
    )(eh, et, nr, hr, ehc, etc_, nrc, hrc)


def kernel(current_triples, corrupted_triples, entity_emb, rel_norm_emb, rel_hyper_emb):
    B = current_triples.shape[0]
    h = current_triples[:, 0]
    r = current_triples[:, 1]
    t = current_triples[:, 2]
    hc = corrupted_triples[:, 0]
    rc = corrupted_triples[:, 1]
    tc = corrupted_triples[:, 2]

    rows = _sc_gather(entity_emb, rel_norm_emb, rel_hyper_emb, h, r, t, hc, rc, tc)
    acc = _tc_loss(*rows)
    rank = acc[0, 0]
    og = acc[0, 1]
    sc = acc[0, 2]
    return rank / B + C * (sc / (4 * B) + og / (2 * B))
